# Initial kernel scaffold; baseline (speedup 1.0000x reference)
import jax
import jax.numpy as jnp
from jax.experimental import pallas as pl


def kernel(hidden_states, routing_weights, expert_select_count, gate_blocks, gate_scales, gate_bias, up_blocks, up_scales, up_bias, down_blocks, down_scales, down_bias):
    raise NotImplementedError("write your pallas kernel here")


# speedup vs baseline: 587.6278x; 587.6278x over previous
import jax
import jax.numpy as jnp
from jax.experimental import pallas as pl

E = 16
H = 1024
I = 1024
T = 64
ALPHA = 1.702
LIMIT = 7.0


def _decode(nib, s):
    sign = (nib >> 3) << 31
    expo = (nib >> 1) & 3
    m = nib & 1
    normal = ((s + expo - 1) << 23) | (m << 22)
    sub = jnp.where(m == 1, (s - 1) << 23, 0)
    bits = jnp.where(expo == 0, sub, normal) | sign
    return jax.lax.bitcast_convert_type(bits, jnp.float32)


def _dequant_cat(bref, sref):
    b = bref[0].astype(jnp.int32)
    s = sref[0].astype(jnp.int32)
    n = b.shape[0]
    s_rep = jnp.broadcast_to(s[:, :, None], (n, 32, 16)).reshape(n, 512)
    lo = _decode(b & 0xF, s_rep)
    hi = _decode(b >> 4, s_rep)
    return jnp.concatenate([lo, hi], axis=1)


def _moe_kernel(x_ref, rw_ref, gb_ref, gs_ref, gbias_ref,
                ub_ref, us_ref, ubias_ref,
                db_ref, ds_ref, dbias_ref, out_ref):
    e = pl.program_id(0)
    x = x_ref[...]
    gw = _dequant_cat(gb_ref, gs_ref)
    g = jax.lax.dot_general(x, gw, (((1,), (1,)), ((), ())),
                            preferred_element_type=jnp.float32)
    g = g + gbias_ref[0]
    uw = _dequant_cat(ub_ref, us_ref)
    u = jax.lax.dot_general(x, uw, (((1,), (1,)), ((), ())),
                            preferred_element_type=jnp.float32)
    u = u + ubias_ref[0]
    g = jnp.minimum(g, LIMIT)
    u = jnp.clip(u, -LIMIT, LIMIT)
    glu = g * jax.nn.sigmoid(ALPHA * g)
    inter = (u + 1.0) * glu
    inter3 = inter.reshape(T, I // 2, 2)
    inter_cat = jnp.concatenate([inter3[:, :, 0], inter3[:, :, 1]], axis=1)
    dw = _dequant_cat(db_ref, ds_ref)
    o = jax.lax.dot_general(inter_cat, dw, (((1,), (1,)), ((), ())),
                            preferred_element_type=jnp.float32)
    o = o + dbias_ref[0]
    lane = jax.lax.broadcasted_iota(jnp.int32, (T, E), 1)
    w_col = jnp.sum(jnp.where(lane == e, rw_ref[...], 0.0),
                    axis=1, keepdims=True)
    contrib = o * w_col

    @pl.when(e == 0)
    def _init():
        out_ref[...] = contrib

    @pl.when(e > 0)
    def _acc():
        out_ref[...] += contrib


def kernel(hidden_states, routing_weights, expert_select_count,
           gate_blocks, gate_scales, gate_bias,
           up_blocks, up_scales, up_bias,
           down_blocks, down_scales, down_bias):
    del expert_select_count
    x_cat = jnp.concatenate(
        [hidden_states[:, 0::2], hidden_states[:, 1::2]], axis=1)
    gbias = gate_bias.reshape(E, 1, I)
    ubias = up_bias.reshape(E, 1, I)
    dbias = down_bias.reshape(E, 1, H)

    def expert_map(e):
        return (e, 0, 0)

    def const_map(e):
        return (0, 0)

    return pl.pallas_call(
        _moe_kernel,
        grid=(E,),
        in_specs=[
            pl.BlockSpec((T, H), const_map),
            pl.BlockSpec((T, E), const_map),
            pl.BlockSpec((1, I, H // 2), expert_map),
            pl.BlockSpec((1, I, H // 32), expert_map),
            pl.BlockSpec((1, 1, I), expert_map),
            pl.BlockSpec((1, I, H // 2), expert_map),
            pl.BlockSpec((1, I, H // 32), expert_map),
            pl.BlockSpec((1, 1, I), expert_map),
            pl.BlockSpec((1, H, I // 2), expert_map),
            pl.BlockSpec((1, H, I // 32), expert_map),
            pl.BlockSpec((1, 1, H), expert_map),
        ],
        out_specs=pl.BlockSpec((T, H), const_map),
        out_shape=jax.ShapeDtypeStruct((T, H), jnp.float32),
    )(x_cat, routing_weights, gate_blocks, gate_scales, gbias,
      up_blocks, up_scales, ubias,
      down_blocks, down_scales, dbias)


# speedup vs baseline: 1619.8152x; 2.7565x over previous
import jax
import jax.numpy as jnp
from jax.experimental import pallas as pl

E = 16
H = 1024
I = 1024
T = 64
ALPHA = 1.702
LIMIT = 7.0


def _decode(nib):
    sign = (nib >> 3) << 31
    expo = (nib >> 1) & 3
    m = nib & 1
    normal = ((126 + expo) << 23) | (m << 22)
    sub = jnp.where(m == 1, 126 << 23, 0)
    bits = jnp.where(expo == 0, sub, normal) | sign
    return jax.lax.bitcast_convert_type(bits, jnp.float32)


def _dequant2(bref, sref):
    b = bref[0].astype(jnp.int32)
    s = sref[0].astype(jnp.int32)
    sc = jax.lax.bitcast_convert_type(s << 23, jnp.float32)
    br = jax.lax.broadcasted_iota(jnp.int32, (32, 512), 0)
    jr = jax.lax.broadcasted_iota(jnp.int32, (32, 512), 1)
    g = ((jr >> 4) == br).astype(jnp.float32)
    s_rep = jax.lax.dot_general(sc, g, (((1,), (0,)), ((), ())),
                                preferred_element_type=jnp.float32)
    return _decode(b & 0xF) * s_rep, _decode(b >> 4) * s_rep


def _dot_t(a, w):
    return jax.lax.dot_general(a, w, (((1,), (1,)), ((), ())),
                               preferred_element_type=jnp.float32)


def _moe_kernel(x_ref, rw_ref, gb_ref, gs_ref, gbias_ref,
                ub_ref, us_ref, ubias_ref,
                db_ref, ds_ref, dbias_ref, out_ref):
    e = pl.program_id(0)
    xe = x_ref[:, :H // 2]
    xo = x_ref[:, H // 2:]
    glo, ghi = _dequant2(gb_ref, gs_ref)
    g = _dot_t(xe, glo) + _dot_t(xo, ghi) + gbias_ref[0]
    ulo, uhi = _dequant2(ub_ref, us_ref)
    u = _dot_t(xe, ulo) + _dot_t(xo, uhi) + ubias_ref[0]
    g = jnp.minimum(g, LIMIT)
    u = jnp.clip(u, -LIMIT, LIMIT)
    glu = g * jax.nn.sigmoid(ALPHA * g)
    inter = (u + 1.0) * glu
    inter3 = inter.reshape(T, I // 2, 2)
    dlo, dhi = _dequant2(db_ref, ds_ref)
    o = _dot_t(inter3[:, :, 0], dlo) + _dot_t(inter3[:, :, 1], dhi)
    o = o + dbias_ref[0]
    lane = jax.lax.broadcasted_iota(jnp.int32, (T, E), 1)
    w_col = jnp.sum(jnp.where(lane == e, rw_ref[...], 0.0),
                    axis=1, keepdims=True)
    contrib = o * w_col

    @pl.when(e == 0)
    def _init():
        out_ref[...] = contrib

    @pl.when(e > 0)
    def _acc():
        out_ref[...] += contrib


def kernel(hidden_states, routing_weights, expert_select_count,
           gate_blocks, gate_scales, gate_bias,
           up_blocks, up_scales, up_bias,
           down_blocks, down_scales, down_bias):
    del expert_select_count
    x_cat = jnp.concatenate(
        [hidden_states[:, 0::2], hidden_states[:, 1::2]], axis=1)
    gbias = gate_bias.reshape(E, 1, I)
    ubias = up_bias.reshape(E, 1, I)
    dbias = down_bias.reshape(E, 1, H)

    def expert_map(e):
        return (e, 0, 0)

    def const_map(e):
        return (0, 0)

    return pl.pallas_call(
        _moe_kernel,
        grid=(E,),
        in_specs=[
            pl.BlockSpec((T, H), const_map),
            pl.BlockSpec((T, E), const_map),
            pl.BlockSpec((1, I, H // 2), expert_map),
            pl.BlockSpec((1, I, H // 32), expert_map),
            pl.BlockSpec((1, 1, I), expert_map),
            pl.BlockSpec((1, I, H // 2), expert_map),
            pl.BlockSpec((1, I, H // 32), expert_map),
            pl.BlockSpec((1, 1, I), expert_map),
            pl.BlockSpec((1, H, I // 2), expert_map),
            pl.BlockSpec((1, H, I // 32), expert_map),
            pl.BlockSpec((1, 1, H), expert_map),
        ],
        out_specs=pl.BlockSpec((T, H), const_map),
        out_shape=jax.ShapeDtypeStruct((T, H), jnp.float32),
    )(x_cat, routing_weights, gate_blocks, gate_scales, gbias,
      up_blocks, up_scales, ubias,
      down_blocks, down_scales, dbias)
